# baseline (speedup 1.0000x reference)
import jax
import jax.numpy as jnp
from jax.experimental import pallas as pl
from jax.experimental.pallas import tpu as pltpu

N = 4096
H = 16
RB = 1024
NI = N // RB
PB = 512
TW = 32


def _pack_body(m_ref, o_ref):
    o_ref[...] = m_ref[...].astype(jnp.int8)


def _pack_call(adj):
    return pl.pallas_call(
        _pack_body,
        grid=(N // PB,),
        in_specs=[pl.BlockSpec((PB, N), lambda i: (i, 0))],
        out_specs=pl.BlockSpec((PB, N), lambda i: (i, 0)),
        out_shape=jax.ShapeDtypeStruct((N, N), jnp.int8),
        compiler_params=pltpu.CompilerParams(
            dimension_semantics=("arbitrary",),
        ),
    )(adj)


def _body(h0_ref, mask_ref, wsw_ref, wsb_ref, wnw_ref, wnb_ref, a2_ref,
          ow_ref, ob_ref, out_ref, h_scr, hself_scr, table_scr, mi8_scr,
          acc_scr):
    l = pl.program_id(0)
    i = pl.program_id(1)

    @pl.when(jnp.logical_and(l == 0, i == 0))
    def _init_h():
        h_scr[...] = h0_ref[...]

    @pl.when(i == 0)
    def _prologue():
        h = h_scr[...]
        h_self = jnp.dot(h, wsw_ref[0],
                         preferred_element_type=jnp.float32) + wsb_ref[0]
        h_n = jnp.dot(h, wnw_ref[0],
                      preferred_element_type=jnp.float32) + wnb_ref[0]
        s2 = jnp.sum(h_n * a2_ref[0], axis=1, keepdims=True)
        e = jnp.exp(jnp.broadcast_to(s2 - jnp.max(s2), (N, H)))
        hself_scr[...] = h_self
        table_scr[...] = jnp.zeros_like(table_scr)
        table_scr[:, 0:H] = (h_n * e).astype(jnp.bfloat16)
        table_scr[:, H:H + 1] = e[:, 0:1].astype(jnp.bfloat16)

    start = i * RB

    @pl.when(l == 0)
    def _dot_from_stream():
        m8 = mask_ref[...]
        mi8_scr[pl.ds(start, RB), :] = m8
        acc_scr[...] = jnp.dot(m8.astype(jnp.bfloat16), table_scr[...],
                               preferred_element_type=jnp.float32)

    @pl.when(l > 0)
    def _dot_from_cache():
        m = mi8_scr[pl.ds(start, RB), :].astype(jnp.bfloat16)
        acc_scr[...] = jnp.dot(m, table_scr[...],
                               preferred_element_type=jnp.float32)

    acc = acc_scr[...]
    num = acc[:, 0:H]
    den = acc[:, H:H + 1]
    den_safe = jnp.where(den > 0.0, den, 1.0)
    h_nb = jnp.where(den > 0.0, num / den_safe, 0.0)
    h_new = jnp.maximum(hself_scr[pl.ds(start, RB), :] + h_nb, 0.0)

    @pl.when(l < 2)
    def _store_h():
        h_scr[pl.ds(start, RB), :] = h_new

    @pl.when(l == 2)
    def _store_out():
        out_ref[...] = jnp.dot(h_new, ow_ref[...],
                               preferred_element_type=jnp.float32) + ob_ref[...]


def kernel(states, adj_matrix, params):
    h0 = jnp.pad(states, ((0, 0), (0, H - states.shape[1])))
    wsw, wsb, wnw, wnb, a2 = [], [], [], [], []
    for l in range(3):
        p = params['l%d' % l]
        w_s, w_n = p['ws_w'].T, p['wn_w'].T
        if w_s.shape[0] < H:
            pad = ((0, H - w_s.shape[0]), (0, 0))
            w_s, w_n = jnp.pad(w_s, pad), jnp.pad(w_n, pad)
        wsw.append(w_s)
        wnw.append(w_n)
        wsb.append(p['ws_b'][None, :])
        wnb.append(p['wn_b'][None, :])
        a2.append(p['a_w'][0, H:][None, :])
    wsw, wnw = jnp.stack(wsw), jnp.stack(wnw)
    wsb, wnb, a2 = jnp.stack(wsb), jnp.stack(wnb), jnp.stack(a2)
    ow = params['out']['w'].T
    ob = params['out']['b'][None, :]
    oc = ob.shape[1]

    mask_i8 = _pack_call(adj_matrix)
    return pl.pallas_call(
        _body,
        grid=(3, NI),
        in_specs=[
            pl.BlockSpec((N, H), lambda l, i: (0, 0)),
            pl.BlockSpec((RB, N),
                         lambda l, i: (jnp.where(l == 0, i, NI - 1), 0)),
            pl.BlockSpec((1, H, H), lambda l, i: (l, 0, 0)),
            pl.BlockSpec((1, 1, H), lambda l, i: (l, 0, 0)),
            pl.BlockSpec((1, H, H), lambda l, i: (l, 0, 0)),
            pl.BlockSpec((1, 1, H), lambda l, i: (l, 0, 0)),
            pl.BlockSpec((1, 1, H), lambda l, i: (l, 0, 0)),
            pl.BlockSpec(ow.shape, lambda l, i: (0, 0)),
            pl.BlockSpec(ob.shape, lambda l, i: (0, 0)),
        ],
        out_specs=pl.BlockSpec((RB, oc),
                               lambda l, i: (jnp.where(l == 2, i, 0), 0)),
        out_shape=jax.ShapeDtypeStruct((N, oc), jnp.float32),
        scratch_shapes=[
            pltpu.VMEM((N, H), jnp.float32),
            pltpu.VMEM((N, H), jnp.float32),
            pltpu.VMEM((N, TW), jnp.bfloat16),
            pltpu.VMEM((N, N), jnp.int8),
            pltpu.VMEM((RB, TW), jnp.float32),
        ],
        compiler_params=pltpu.CompilerParams(
            dimension_semantics=("arbitrary", "arbitrary"),
        ),
    )(h0, mask_i8, wsw, wsb, wnw, wnb, a2, ow, ob)


# speedup vs baseline: 1.1840x; 1.1840x over previous
import functools

import jax
import jax.numpy as jnp
from jax.experimental import pallas as pl
from jax.experimental.pallas import tpu as pltpu

N = 4096
H = 16
TW = 32


def _layer_body(h_ref, mask_ref, wsw_ref, wsb_ref, wnw_ref, wnb_ref, a2_ref,
                ow_ref, ob_ref, *refs, rb, first, final):
    if first:
        out_ref, i8_ref, hself_scr, table_scr = refs
    else:
        out_ref, hself_scr, table_scr = refs
        i8_ref = None
    i = pl.program_id(0)

    @pl.when(i == 0)
    def _prologue():
        h = h_ref[...]
        h_self = jnp.dot(h, wsw_ref[...],
                         preferred_element_type=jnp.float32) + wsb_ref[...]
        h_n = jnp.dot(h, wnw_ref[...],
                      preferred_element_type=jnp.float32) + wnb_ref[...]
        s2 = jnp.sum(h_n * a2_ref[...], axis=1, keepdims=True)
        e = jnp.exp(jnp.broadcast_to(s2 - jnp.max(s2), (N, H)))
        hself_scr[...] = h_self
        table_scr[...] = jnp.zeros_like(table_scr)
        table_scr[:, 0:H] = (h_n * e).astype(jnp.bfloat16)
        table_scr[:, H:H + 1] = e[:, 0:1].astype(jnp.bfloat16)

    if first:
        m32 = mask_ref[...]
        i8_ref[...] = m32.astype(jnp.int8)
        m = m32.astype(jnp.bfloat16)
    else:
        m = mask_ref[...].astype(jnp.bfloat16)
    acc = jnp.dot(m, table_scr[...],
                  preferred_element_type=jnp.float32)
    num = acc[:, 0:H]
    den = acc[:, H:H + 1]
    den_safe = jnp.where(den > 0.0, den, 1.0)
    h_nb = jnp.where(den > 0.0, num / den_safe, 0.0)
    hs = hself_scr[pl.ds(i * rb, rb), :]
    h_new = jnp.maximum(hs + h_nb, 0.0)
    if final:
        out_ref[...] = jnp.dot(h_new, ow_ref[...],
                               preferred_element_type=jnp.float32) + ob_ref[...]
    else:
        out_ref[...] = h_new


def _layer_call(h, adj, wsw, wsb, wnw, wnb, a2, ow, ob, *, rb, first, final):
    out_cols = ob.shape[1] if final else H
    out_shape = [jax.ShapeDtypeStruct((N, out_cols), jnp.float32)]
    out_specs = [pl.BlockSpec((rb, out_cols), lambda i: (i, 0))]
    if first:
        out_shape.append(jax.ShapeDtypeStruct((N, N), jnp.int8))
        out_specs.append(pl.BlockSpec((rb, N), lambda i: (i, 0)))
    res = pl.pallas_call(
        functools.partial(_layer_body, rb=rb, first=first, final=final),
        grid=(N // rb,),
        in_specs=[
            pl.BlockSpec((N, H), lambda i: (0, 0)),
            pl.BlockSpec((rb, N), lambda i: (i, 0)),
            pl.BlockSpec((H, H), lambda i: (0, 0)),
            pl.BlockSpec((1, H), lambda i: (0, 0)),
            pl.BlockSpec((H, H), lambda i: (0, 0)),
            pl.BlockSpec((1, H), lambda i: (0, 0)),
            pl.BlockSpec((1, H), lambda i: (0, 0)),
            pl.BlockSpec(ow.shape, lambda i: (0, 0)),
            pl.BlockSpec(ob.shape, lambda i: (0, 0)),
        ],
        out_specs=out_specs,
        out_shape=out_shape,
        scratch_shapes=[
            pltpu.VMEM((N, H), jnp.float32),
            pltpu.VMEM((N, TW), jnp.bfloat16),
        ],
        compiler_params=pltpu.CompilerParams(
            dimension_semantics=("arbitrary",),
        ),
    )(h, adj, wsw, wsb, wnw, wnb, a2, ow, ob)
    return res


RB12 = 2048
NI12 = N // RB12


def _l12_body(h1_ref, mask_ref, wsw_ref, wsb_ref, wnw_ref, wnb_ref, a2_ref,
              ow_ref, ob_ref, out_ref, h_scr, hself_scr, table_scr, mi8_scr):
    l = pl.program_id(0)
    i = pl.program_id(1)

    @pl.when(jnp.logical_and(l == 0, i == 0))
    def _init_h():
        h_scr[...] = h1_ref[...]

    @pl.when(i == 0)
    def _prologue():
        h = h_scr[...]
        h_self = jnp.dot(h, wsw_ref[0],
                         preferred_element_type=jnp.float32) + wsb_ref[0]
        h_n = jnp.dot(h, wnw_ref[0],
                      preferred_element_type=jnp.float32) + wnb_ref[0]
        s2 = jnp.sum(h_n * a2_ref[0], axis=1, keepdims=True)
        e = jnp.exp(jnp.broadcast_to(s2 - jnp.max(s2), (N, H)))
        hself_scr[...] = h_self
        table_scr[...] = jnp.zeros_like(table_scr)
        table_scr[:, 0:H] = (h_n * e).astype(jnp.bfloat16)
        table_scr[:, H:H + 1] = e[:, 0:1].astype(jnp.bfloat16)

    start = i * RB12

    @pl.when(l == 0)
    def _cache_mask():
        mi8_scr[pl.ds(start, RB12), :] = mask_ref[...]

    m = mi8_scr[pl.ds(start, RB12), :].astype(jnp.bfloat16)
    acc = jnp.dot(m, table_scr[...],
                  preferred_element_type=jnp.float32)
    num = acc[:, 0:H]
    den = acc[:, H:H + 1]
    den_safe = jnp.where(den > 0.0, den, 1.0)
    h_nb = jnp.where(den > 0.0, num / den_safe, 0.0)
    h_new = jnp.maximum(hself_scr[pl.ds(start, RB12), :] + h_nb, 0.0)

    @pl.when(l == 0)
    def _store_h():
        h_scr[pl.ds(start, RB12), :] = h_new

    @pl.when(l == 1)
    def _store_out():
        out_ref[...] = jnp.dot(h_new, ow_ref[...],
                               preferred_element_type=jnp.float32) + ob_ref[...]


def _l12_call(h1, mask_i8, wsw, wsb, wnw, wnb, a2, ow, ob):
    oc = ob.shape[1]
    return pl.pallas_call(
        _l12_body,
        grid=(2, NI12),
        in_specs=[
            pl.BlockSpec((N, H), lambda l, i: (0, 0)),
            pl.BlockSpec((RB12, N),
                         lambda l, i: (jnp.where(l == 0, i, NI12 - 1), 0)),
            pl.BlockSpec((1, H, H), lambda l, i: (l, 0, 0)),
            pl.BlockSpec((1, 1, H), lambda l, i: (l, 0, 0)),
            pl.BlockSpec((1, H, H), lambda l, i: (l, 0, 0)),
            pl.BlockSpec((1, 1, H), lambda l, i: (l, 0, 0)),
            pl.BlockSpec((1, 1, H), lambda l, i: (l, 0, 0)),
            pl.BlockSpec(ow.shape, lambda l, i: (0, 0)),
            pl.BlockSpec(ob.shape, lambda l, i: (0, 0)),
        ],
        out_specs=pl.BlockSpec((RB12, oc),
                               lambda l, i: (jnp.where(l == 1, i, 0), 0)),
        out_shape=jax.ShapeDtypeStruct((N, oc), jnp.float32),
        scratch_shapes=[
            pltpu.VMEM((N, H), jnp.float32),
            pltpu.VMEM((N, H), jnp.float32),
            pltpu.VMEM((N, TW), jnp.bfloat16),
            pltpu.VMEM((N, N), jnp.int8),
        ],
        compiler_params=pltpu.CompilerParams(
            dimension_semantics=("arbitrary", "arbitrary"),
        ),
    )(h1, mask_i8, wsw, wsb, wnw, wnb, a2, ow, ob)


def _layer_params(p, in_dim):
    wsw, wnw = p['ws_w'].T, p['wn_w'].T
    if in_dim < H:
        pad = ((0, H - in_dim), (0, 0))
        wsw, wnw = jnp.pad(wsw, pad), jnp.pad(wnw, pad)
    return wsw, p['ws_b'][None, :], wnw, p['wn_b'][None, :], p['a_w'][0, H:][None, :]


def kernel(states, adj_matrix, params):
    h = jnp.pad(states, ((0, 0), (0, H - states.shape[1])))
    ow = params['out']['w'].T
    ob = params['out']['b'][None, :]

    wsw0, wsb0, wnw0, wnb0, a20 = _layer_params(params['l0'],
                                                params['l0']['ws_w'].shape[1])
    h1, mask_i8 = _layer_call(h, adj_matrix, wsw0, wsb0, wnw0, wnb0, a20,
                              ow, ob, rb=512, first=True, final=False)

    ps = [_layer_params(params['l%d' % l], H) for l in (1, 2)]
    wsw = jnp.stack([p[0] for p in ps])
    wsb = jnp.stack([p[1] for p in ps])
    wnw = jnp.stack([p[2] for p in ps])
    wnb = jnp.stack([p[3] for p in ps])
    a2 = jnp.stack([p[4] for p in ps])
    return _l12_call(h1, mask_i8, wsw, wsb, wnw, wnb, a2, ow, ob)


# speedup vs baseline: 1.2077x; 1.0200x over previous
import functools

import jax
import jax.numpy as jnp
from jax.experimental import pallas as pl
from jax.experimental.pallas import tpu as pltpu

N = 4096
H = 16
TW = 32


def _layer_body(h_ref, mask_ref, wsw_ref, wsb_ref, wnw_ref, wnb_ref, a2_ref,
                ow_ref, ob_ref, *refs, rb, first, final):
    if first:
        out_ref, i8_ref, hself_scr, table_scr = refs
    else:
        out_ref, hself_scr, table_scr = refs
        i8_ref = None
    i = pl.program_id(0)

    @pl.when(i == 0)
    def _prologue():
        h = h_ref[...]
        h_self = jnp.dot(h, wsw_ref[...],
                         preferred_element_type=jnp.float32) + wsb_ref[...]
        h_n = jnp.dot(h, wnw_ref[...],
                      preferred_element_type=jnp.float32) + wnb_ref[...]
        s2 = jnp.sum(h_n * a2_ref[...], axis=1, keepdims=True)
        e = jnp.exp(jnp.broadcast_to(s2 - jnp.max(s2), (N, H)))
        hself_scr[...] = h_self
        table_scr[...] = jnp.zeros_like(table_scr)
        table_scr[:, 0:H] = h_n * e
        table_scr[:, H:H + 1] = e[:, 0:1]

    if first:
        m = mask_ref[...]
        i8_ref[...] = m.astype(jnp.int8)
    else:
        m = mask_ref[...]
    acc = jnp.dot(m, table_scr[...],
                  preferred_element_type=jnp.float32)
    num = acc[:, 0:H]
    den = acc[:, H:H + 1]
    den_safe = jnp.where(den > 0.0, den, 1.0)
    h_nb = jnp.where(den > 0.0, num / den_safe, 0.0)
    hs = hself_scr[pl.ds(i * rb, rb), :]
    h_new = jnp.maximum(hs + h_nb, 0.0)
    if final:
        out_ref[...] = jnp.dot(h_new, ow_ref[...],
                               preferred_element_type=jnp.float32) + ob_ref[...]
    else:
        out_ref[...] = h_new


def _layer_call(h, adj, wsw, wsb, wnw, wnb, a2, ow, ob, *, rb, first, final):
    out_cols = ob.shape[1] if final else H
    out_shape = [jax.ShapeDtypeStruct((N, out_cols), jnp.float32)]
    out_specs = [pl.BlockSpec((rb, out_cols), lambda i: (i, 0))]
    if first:
        out_shape.append(jax.ShapeDtypeStruct((N, N), jnp.int8))
        out_specs.append(pl.BlockSpec((rb, N), lambda i: (i, 0)))
    res = pl.pallas_call(
        functools.partial(_layer_body, rb=rb, first=first, final=final),
        grid=(N // rb,),
        in_specs=[
            pl.BlockSpec((N, H), lambda i: (0, 0)),
            pl.BlockSpec((rb, N), lambda i: (i, 0)),
            pl.BlockSpec((H, H), lambda i: (0, 0)),
            pl.BlockSpec((1, H), lambda i: (0, 0)),
            pl.BlockSpec((H, H), lambda i: (0, 0)),
            pl.BlockSpec((1, H), lambda i: (0, 0)),
            pl.BlockSpec((1, H), lambda i: (0, 0)),
            pl.BlockSpec(ow.shape, lambda i: (0, 0)),
            pl.BlockSpec(ob.shape, lambda i: (0, 0)),
        ],
        out_specs=out_specs,
        out_shape=out_shape,
        scratch_shapes=[
            pltpu.VMEM((N, H), jnp.float32),
            pltpu.VMEM((N, TW), jnp.float32),
        ],
        compiler_params=pltpu.CompilerParams(
            dimension_semantics=("arbitrary",),
        ),
    )(h, adj, wsw, wsb, wnw, wnb, a2, ow, ob)
    return res


RB12 = 1024
NI12 = N // RB12


def _l12_body(h1_ref, mask_ref, wsw_ref, wsb_ref, wnw_ref, wnb_ref, a2_ref,
              ow_ref, ob_ref, out_ref, h_scr, hself_scr, table_scr, mi8_scr):
    l = pl.program_id(0)
    i = pl.program_id(1)

    @pl.when(jnp.logical_and(l == 0, i == 0))
    def _init_h():
        h_scr[...] = h1_ref[...]

    @pl.when(i == 0)
    def _prologue():
        h = h_scr[...]
        h_self = jnp.dot(h, wsw_ref[0],
                         preferred_element_type=jnp.float32) + wsb_ref[0]
        h_n = jnp.dot(h, wnw_ref[0],
                      preferred_element_type=jnp.float32) + wnb_ref[0]
        s2 = jnp.sum(h_n * a2_ref[0], axis=1, keepdims=True)
        e = jnp.exp(jnp.broadcast_to(s2 - jnp.max(s2), (N, H)))
        hself_scr[...] = h_self
        table_scr[...] = jnp.zeros_like(table_scr)
        table_scr[:, 0:H] = (h_n * e).astype(jnp.bfloat16)
        table_scr[:, H:H + 1] = e[:, 0:1].astype(jnp.bfloat16)

    start = i * RB12

    @pl.when(l == 0)
    def _cache_mask():
        mi8_scr[pl.ds(start, RB12), :] = mask_ref[...]

    m = mi8_scr[pl.ds(start, RB12), :].astype(jnp.bfloat16)
    acc = jnp.dot(m, table_scr[...],
                  preferred_element_type=jnp.float32)
    num = acc[:, 0:H]
    den = acc[:, H:H + 1]
    den_safe = jnp.where(den > 0.0, den, 1.0)
    h_nb = jnp.where(den > 0.0, num / den_safe, 0.0)
    h_new = jnp.maximum(hself_scr[pl.ds(start, RB12), :] + h_nb, 0.0)

    @pl.when(l == 0)
    def _store_h():
        h_scr[pl.ds(start, RB12), :] = h_new

    @pl.when(l == 1)
    def _store_out():
        out_ref[...] = jnp.dot(h_new, ow_ref[...],
                               preferred_element_type=jnp.float32) + ob_ref[...]


def _l12_call(h1, mask_i8, wsw, wsb, wnw, wnb, a2, ow, ob):
    oc = ob.shape[1]
    return pl.pallas_call(
        _l12_body,
        grid=(2, NI12),
        in_specs=[
            pl.BlockSpec((N, H), lambda l, i: (0, 0)),
            pl.BlockSpec((RB12, N),
                         lambda l, i: (jnp.where(l == 0, i, NI12 - 1), 0)),
            pl.BlockSpec((1, H, H), lambda l, i: (l, 0, 0)),
            pl.BlockSpec((1, 1, H), lambda l, i: (l, 0, 0)),
            pl.BlockSpec((1, H, H), lambda l, i: (l, 0, 0)),
            pl.BlockSpec((1, 1, H), lambda l, i: (l, 0, 0)),
            pl.BlockSpec((1, 1, H), lambda l, i: (l, 0, 0)),
            pl.BlockSpec(ow.shape, lambda l, i: (0, 0)),
            pl.BlockSpec(ob.shape, lambda l, i: (0, 0)),
        ],
        out_specs=pl.BlockSpec((RB12, oc),
                               lambda l, i: (jnp.where(l == 1, i, 0), 0)),
        out_shape=jax.ShapeDtypeStruct((N, oc), jnp.float32),
        scratch_shapes=[
            pltpu.VMEM((N, H), jnp.float32),
            pltpu.VMEM((N, H), jnp.float32),
            pltpu.VMEM((N, TW), jnp.bfloat16),
            pltpu.VMEM((N, N), jnp.int8),
        ],
        compiler_params=pltpu.CompilerParams(
            dimension_semantics=("arbitrary", "arbitrary"),
        ),
    )(h1, mask_i8, wsw, wsb, wnw, wnb, a2, ow, ob)


def _layer_params(p, in_dim):
    wsw, wnw = p['ws_w'].T, p['wn_w'].T
    if in_dim < H:
        pad = ((0, H - in_dim), (0, 0))
        wsw, wnw = jnp.pad(wsw, pad), jnp.pad(wnw, pad)
    return wsw, p['ws_b'][None, :], wnw, p['wn_b'][None, :], p['a_w'][0, H:][None, :]


def kernel(states, adj_matrix, params):
    h = jnp.pad(states, ((0, 0), (0, H - states.shape[1])))
    ow = params['out']['w'].T
    ob = params['out']['b'][None, :]

    wsw0, wsb0, wnw0, wnb0, a20 = _layer_params(params['l0'],
                                                params['l0']['ws_w'].shape[1])
    h1, mask_i8 = _layer_call(h, adj_matrix, wsw0, wsb0, wnw0, wnb0, a20,
                              ow, ob, rb=512, first=True, final=False)

    ps = [_layer_params(params['l%d' % l], H) for l in (1, 2)]
    wsw = jnp.stack([p[0] for p in ps])
    wsb = jnp.stack([p[1] for p in ps])
    wnw = jnp.stack([p[2] for p in ps])
    wnb = jnp.stack([p[3] for p in ps])
    a2 = jnp.stack([p[4] for p in ps])
    return _l12_call(h1, mask_i8, wsw, wsb, wnw, wnb, a2, ow, ob)
